# baseline (speedup 1.0000x reference)
import functools

import jax
import jax.numpy as jnp
from jax import lax
from jax.experimental import pallas as pl
from jax.experimental.pallas import tpu as pltpu
from jax.experimental.pallas import tpu_sc as plsc

_B = 16384
_L = 50
_D = 64
_LANES = 16
_CHUNK = 128


def _sc_workers():
    try:
        info = plsc.get_sparse_core_info()
        return info.num_cores, info.num_subcores
    except Exception:
        return 2, 16


@functools.partial(jax.jit, static_argnames=("nc", "ns"))
def _trans_rec(user, item_seq_flat, item_seq_len, user_table, item_table, T,
               nc, ns):
    nw = nc * ns
    b_per_w = _B // nw
    n_chunks = b_per_w // _CHUNK
    mesh = plsc.VectorSubcoreMesh(
        core_axis_name="c", subcore_axis_name="s", num_cores=nc,
        num_subcores=ns)

    @functools.partial(
        pl.kernel,
        out_type=jax.ShapeDtypeStruct((_B, _D), jnp.float32),
        mesh=mesh,
        compiler_params=pltpu.CompilerParams(use_tc_tiling_on_sc=False),
        scratch_types=[
            pltpu.VMEM((b_per_w,), jnp.int32),
            pltpu.VMEM((b_per_w,), jnp.int32),
            pltpu.VMEM((b_per_w,), jnp.int32),
            pltpu.VMEM((b_per_w,), jnp.int32),
            pltpu.VMEM((b_per_w, _D), jnp.float32),
            pltpu.VMEM((_D,), jnp.float32),
            pltpu.SemaphoreType.DMA,
            pltpu.SemaphoreType.DMA,
        ],
    )
    def body(user_hbm, iseq_hbm, len_hbm, utab_hbm, itab_hbm, t_hbm,
             out_hbm, uidx_v, len_v, fidx_v, last_v, acc_v, t_v, sem_a,
             sem_b):
        wid = lax.axis_index("s") * nc + lax.axis_index("c")
        base = wid * b_per_w

        pltpu.sync_copy(user_hbm.at[pl.ds(base, b_per_w)], uidx_v)
        pltpu.sync_copy(len_hbm.at[pl.ds(base, b_per_w)], len_v)
        pltpu.sync_copy(t_hbm, t_v)

        for j in range(b_per_w // _LANES):
            seq_len = len_v[pl.ds(j * _LANES, _LANES)]
            row = lax.iota(jnp.int32, _LANES) + (base + j * _LANES)
            fidx_v[pl.ds(j * _LANES, _LANES)] = row * _L + seq_len - 1

        copies = []
        for k in range(n_chunks):
            sl = pl.ds(k * _CHUNK, _CHUNK)
            copies.append(pltpu.async_copy(
                iseq_hbm.at[fidx_v.at[sl]], last_v.at[sl], sem_a))
            copies.append(pltpu.async_copy(
                utab_hbm.at[uidx_v.at[sl]], acc_v.at[sl], sem_b))
        for c in copies:
            c.wait()

        copies = []
        for k in range(n_chunks):
            sl = pl.ds(k * _CHUNK, _CHUNK)
            copies.append(pltpu.async_copy(
                itab_hbm.at[last_v.at[sl]], acc_v.at[sl], sem_b, add=True))
        for c in copies:
            c.wait()

        t_regs = [t_v[pl.ds(d * _LANES, _LANES)] for d in range(_D // _LANES)]

        def add_t(g, _):
            for d in range(_D // _LANES):
                sl = pl.ds(d * _LANES, _LANES)
                acc_v[g, sl] = acc_v[g, sl] + t_regs[d]
            return 0

        lax.fori_loop(0, b_per_w, add_t, 0)

        pltpu.sync_copy(acc_v, out_hbm.at[pl.ds(base, b_per_w)])

    return body(user, item_seq_flat, item_seq_len, user_table, item_table, T)


def kernel(user, item_seq, item_seq_len, user_table, item_table, T):
    nc, ns = _sc_workers()
    return _trans_rec(
        user.astype(jnp.int32),
        item_seq.reshape(-1).astype(jnp.int32),
        item_seq_len.astype(jnp.int32),
        user_table, item_table, T, nc, ns)
